# Initial kernel scaffold; baseline (speedup 1.0000x reference)
import jax
import jax.numpy as jnp
from jax.experimental import pallas as pl


def kernel(tokens, table, W, b):
    raise NotImplementedError("write your pallas kernel here")


# speedup vs baseline: 3.3127x; 3.3127x over previous
import functools

import jax
import jax.numpy as jnp
from jax import lax
from jax.experimental import pallas as pl
from jax.experimental.pallas import tpu as pltpu
from jax.experimental.pallas import tpu_sc as plsc

_B = 4096
_L = 200
_EMB = 64
_OUT = 128

_NW = 32
_CH = 104
_LP = 104
_NBUF = 4
_CHUNKS = 2 * _B // _NW
_NGROUP = _CHUNKS // _NBUF

_BB = 128
_SENT = -1e35


def _sc_gather(tok2, table):
    mesh = plsc.VectorSubcoreMesh(core_axis_name="c", subcore_axis_name="s")

    @functools.partial(
        pl.kernel,
        mesh=mesh,
        out_type=jax.ShapeDtypeStruct((_B, _LP, 2 * _EMB), jnp.float32),
        scratch_types=[
            pltpu.VMEM((_CHUNKS, _CH), jnp.int32),
            pltpu.VMEM((_NBUF, _CH, _EMB), jnp.float32),
            pltpu.SemaphoreType.DMA,
            pltpu.SemaphoreType.DMA,
            pltpu.SemaphoreType.DMA,
            pltpu.SemaphoreType.DMA,
        ],
        compiler_params=pltpu.CompilerParams(use_tc_tiling_on_sc=False),
    )
    def k(tok_hbm, table_hbm, out_hbm, idx_v, g_v, s0, s1, s2, s3):
        sems = [s0, s1, s2, s3]
        wid = lax.axis_index("s") * 2 + lax.axis_index("c")
        pltpu.sync_copy(tok_hbm.at[pl.ds(wid * _CHUNKS, _CHUNKS)], idx_v)
        for t in range(_NBUF):
            pltpu.async_copy(table_hbm.at[idx_v.at[t]], g_v.at[t], sems[t])

        def body(g, carry):
            for t in range(_NBUF):
                j = g * _NBUF + t
                pltpu.make_async_copy(
                    table_hbm.at[idx_v.at[j]], g_v.at[t], sems[t]
                ).wait()
                b_idx = wid * (_CHUNKS // 2) + g * (_NBUF // 2) + t // 2
                h = t % 2
                pltpu.sync_copy(
                    g_v.at[t, pl.ds(0, _L // 2)],
                    out_hbm.at[b_idx].at[pl.ds(0, _L // 2),
                                         pl.ds(_EMB * h, _EMB)],
                )

                @pl.when(g < _NGROUP - 1)
                def _():
                    pltpu.async_copy(
                        table_hbm.at[idx_v.at[j + _NBUF]], g_v.at[t], sems[t]
                    )

            return carry

        lax.fori_loop(0, _NGROUP, body, 0)

    return k(tok2, table)


def _tc_body(x_ref, wt_ref, b_ref, o_ref):
    xr = x_ref[...]
    sent = jnp.float32(_SENT)
    pad = lax.broadcasted_iota(jnp.int32, (_BB, _LP, 2 * _EMB), 1) >= _L // 2
    x0 = jnp.where(pad, sent, xr)

    def fold(op, a128):
        return op(a128[:, :_EMB], a128[:, _EMB:])

    def unfold(a64):
        return jnp.concatenate([a64, a64], axis=-1)[:, None, :]

    def round_(_, carry):
        xw, m, c, s_prev, v20, v21 = carry
        xw = jnp.where(xw >= unfold(m), sent, xw)
        s = fold(jnp.add, jnp.sum(xw, axis=1))
        k = jnp.round((s_prev - s) * jnp.float32(1e-35))
        newc = c + k
        v20 = jnp.where((c < 20.0) & (newc >= 20.0), m, v20)
        v21 = jnp.where((c < 21.0) & (newc >= 21.0), m, v21)
        m = fold(jnp.maximum, jnp.max(xw, axis=1))
        return xw, m, newc, s, v20, v21

    zeros = jnp.zeros((_BB, _EMB), jnp.float32)
    m0 = fold(jnp.maximum, jnp.max(x0, axis=1))
    s0 = fold(jnp.add, jnp.sum(x0, axis=1))
    _, _, _, _, v20, v21 = lax.fori_loop(
        0, 21, round_, (x0, m0, zeros, s0, zeros, zeros)
    )
    qs = v21 + jnp.float32(0.1) * (v20 - v21)
    s128 = jnp.sum(jnp.where(x0 >= unfold(qs)[:, 0, :][:, None, :], x0, 0.0),
                   axis=1)
    pooled = fold(jnp.add, s128) * jnp.float32(1.0 / _L)
    o_ref[...] = (
        jnp.dot(pooled, wt_ref[...], preferred_element_type=jnp.float32)
        + b_ref[...]
    )


def _tc_call(gathered3, wt, b2):
    grid = _B // _BB
    return pl.pallas_call(
        _tc_body,
        grid=(grid,),
        in_specs=[
            pl.BlockSpec((_BB, _LP, 2 * _EMB), lambda i: (i, 0, 0)),
            pl.BlockSpec((_EMB, _OUT), lambda i: (0, 0)),
            pl.BlockSpec((1, _OUT), lambda i: (0, 0)),
        ],
        out_specs=pl.BlockSpec((_BB, _OUT), lambda i: (i, 0)),
        out_shape=jax.ShapeDtypeStruct((_B, _OUT), jnp.float32),
    )(gathered3, wt, b2)


def kernel(tokens, table, W, b):
    tok_r = tokens.astype(jnp.int32).reshape(_B, _L // 2, 2).transpose(0, 2, 1)
    tok_p = jnp.pad(tok_r, ((0, 0), (0, 0), (0, _CH - _L // 2)))
    tok2 = tok_p.reshape(2 * _B, _CH)
    gathered3 = _sc_gather(tok2, table)
    wt = W.T
    b2 = b.reshape(1, _OUT)
    return _tc_call(gathered3, wt, b2)


# speedup vs baseline: 5.3524x; 1.6157x over previous
import functools

import jax
import jax.numpy as jnp
from jax import lax
from jax.experimental import pallas as pl
from jax.experimental.pallas import tpu as pltpu
from jax.experimental.pallas import tpu_sc as plsc

_B = 4096
_L = 200
_EMB = 64
_OUT = 128

_NW = 32
_CH = 128
_NBUF = 4
_CHUNKS = (_B * _L) // (_NW * _CH)
_NGROUP = _CHUNKS // _NBUF

_BB = 128
_LH = _L // 2
_SENT = -1e35


def _sc_gather(tok2, table):
    mesh = plsc.VectorSubcoreMesh(core_axis_name="c", subcore_axis_name="s")

    @functools.partial(
        pl.kernel,
        mesh=mesh,
        out_type=jax.ShapeDtypeStruct((_B * _L, _EMB), jnp.float32),
        scratch_types=[
            pltpu.VMEM((_CHUNKS, _CH), jnp.int32),
            pltpu.VMEM((_NBUF, _CH, _EMB), jnp.float32),
            pltpu.SemaphoreType.DMA,
            pltpu.SemaphoreType.DMA,
            pltpu.SemaphoreType.DMA,
            pltpu.SemaphoreType.DMA,
        ],
        compiler_params=pltpu.CompilerParams(use_tc_tiling_on_sc=False),
    )
    def k(tok_hbm, table_hbm, out_hbm, idx_v, rows_v, s0, s1, s2, s3):
        sems = [s0, s1, s2, s3]
        wid = lax.axis_index("s") * 2 + lax.axis_index("c")
        pltpu.sync_copy(tok_hbm.at[pl.ds(wid * _CHUNKS, _CHUNKS)], idx_v)
        for t in range(_NBUF):
            pltpu.async_copy(table_hbm.at[idx_v.at[t]], rows_v.at[t], sems[t])

        def body(g, carry):
            for t in range(_NBUF):
                j = g * _NBUF + t
                pltpu.make_async_copy(
                    table_hbm.at[idx_v.at[j]], rows_v.at[t], sems[t]
                ).wait()
                pltpu.sync_copy(
                    rows_v.at[t],
                    out_hbm.at[pl.ds((wid * _CHUNKS + j) * _CH, _CH)],
                )

                @pl.when(g < _NGROUP - 1)
                def _():
                    pltpu.async_copy(
                        table_hbm.at[idx_v.at[j + _NBUF]], rows_v.at[t], sems[t]
                    )

            return carry

        lax.fori_loop(0, _NGROUP, body, 0)

    return k(tok2, table)


def _tc_body(x_ref, wt_ref, b_ref, o_ref):
    x0 = x_ref[...]
    sent = jnp.float32(_SENT)

    def fold(op, a128):
        return op(a128[:, :_EMB], a128[:, _EMB:])

    def unfold(a64):
        return jnp.concatenate([a64, a64], axis=-1)[:, None, :]

    def round_(_, carry):
        t, c, v20, v21 = carry
        masked = jnp.where(x0 < unfold(t), x0, sent)
        m = fold(jnp.maximum, jnp.max(masked, axis=1))
        cnt = fold(jnp.add,
                   jnp.sum(jnp.where(x0 >= unfold(m), 1.0, 0.0), axis=1))
        v20 = jnp.where((c < 20.0) & (cnt >= 20.0), m, v20)
        v21 = jnp.where((c < 21.0) & (cnt >= 21.0), m, v21)
        return m, cnt, v20, v21

    zeros = jnp.zeros((_BB, _EMB), jnp.float32)
    inf = jnp.full((_BB, _EMB), jnp.inf, jnp.float32)
    _, _, v20, v21 = lax.fori_loop(
        0, 21, round_, (inf, zeros, zeros, zeros)
    )
    qs = v21 + jnp.float32(0.1) * (v20 - v21)
    s128 = jnp.sum(jnp.where(x0 >= unfold(qs), x0, 0.0), axis=1)
    pooled = fold(jnp.add, s128) * jnp.float32(1.0 / _L)
    o_ref[...] = (
        jnp.dot(pooled, wt_ref[...], preferred_element_type=jnp.float32)
        + b_ref[...]
    )


def _tc_call(gathered3, wt, b2):
    grid = _B // _BB
    return pl.pallas_call(
        _tc_body,
        grid=(grid,),
        in_specs=[
            pl.BlockSpec((_BB, _LH, 2 * _EMB), lambda i: (i, 0, 0)),
            pl.BlockSpec((_EMB, _OUT), lambda i: (0, 0)),
            pl.BlockSpec((1, _OUT), lambda i: (0, 0)),
        ],
        out_specs=pl.BlockSpec((_BB, _OUT), lambda i: (i, 0)),
        out_shape=jax.ShapeDtypeStruct((_B, _OUT), jnp.float32),
    )(gathered3, wt, b2)


def kernel(tokens, table, W, b):
    tok2 = tokens.astype(jnp.int32).reshape(-1, _CH)
    gathered = _sc_gather(tok2, table)
    gathered3 = gathered.reshape(_B, _LH, 2 * _EMB)
    wt = W.T
    b2 = b.reshape(1, _OUT)
    return _tc_call(gathered3, wt, b2)


# speedup vs baseline: 5.8260x; 1.0885x over previous
import functools

import jax
import jax.numpy as jnp
from jax import lax
from jax.experimental import pallas as pl
from jax.experimental.pallas import tpu as pltpu
from jax.experimental.pallas import tpu_sc as plsc

_B = 4096
_L = 200
_EMB = 64
_OUT = 128

_NW = 32
_CH = 128
_NBUF = 4
_CHUNKS = (_B * _L) // (_NW * _CH)
_NGROUP = _CHUNKS // _NBUF

_BB = 128
_LH = _L // 2
_SENT = -1e35


def _sc_gather(tok2, table):
    mesh = plsc.VectorSubcoreMesh(core_axis_name="c", subcore_axis_name="s")

    @functools.partial(
        pl.kernel,
        mesh=mesh,
        out_type=jax.ShapeDtypeStruct((_B * _L, _EMB), jnp.float32),
        scratch_types=[
            pltpu.VMEM((_CHUNKS, _CH), jnp.int32),
            pltpu.VMEM((_NBUF, _CH, _EMB), jnp.float32),
            pltpu.SemaphoreType.DMA,
            pltpu.SemaphoreType.DMA,
            pltpu.SemaphoreType.DMA,
            pltpu.SemaphoreType.DMA,
        ],
        compiler_params=pltpu.CompilerParams(use_tc_tiling_on_sc=False),
    )
    def k(tok_hbm, table_hbm, out_hbm, idx_v, rows_v, s0, s1, s2, s3):
        sems = [s0, s1, s2, s3]
        wid = lax.axis_index("s") * 2 + lax.axis_index("c")
        pltpu.sync_copy(tok_hbm.at[pl.ds(wid * _CHUNKS, _CHUNKS)], idx_v)
        for t in range(_NBUF):
            pltpu.async_copy(table_hbm.at[idx_v.at[t]], rows_v.at[t], sems[t])

        def body(g, carry):
            for t in range(_NBUF):
                j = g * _NBUF + t
                pltpu.make_async_copy(
                    table_hbm.at[idx_v.at[j]], rows_v.at[t], sems[t]
                ).wait()
                pltpu.sync_copy(
                    rows_v.at[t],
                    out_hbm.at[pl.ds((wid * _CHUNKS + j) * _CH, _CH)],
                )

                @pl.when(g < _NGROUP - 1)
                def _():
                    pltpu.async_copy(
                        table_hbm.at[idx_v.at[j + _NBUF]], rows_v.at[t], sems[t]
                    )

            return carry

        lax.fori_loop(0, _NGROUP, body, 0)

    return k(tok2, table)


def _tc_body(x_ref, wt_ref, b_ref, o_ref):
    x0 = x_ref[...]
    sent = jnp.float32(_SENT)

    def dup(op, a128):
        return op(a128, jnp.roll(a128, _EMB, axis=-1))

    def round_(_, carry):
        t, c, v20, v21 = carry
        masked = jnp.where(x0 < t[:, None, :], x0, sent)
        m = dup(jnp.maximum, jnp.max(masked, axis=1))
        s = dup(jnp.add, jnp.sum(masked, axis=1))
        cnt = jnp.round(s * jnp.float32(-1e-35))
        v20 = jnp.where((c < 20.0) & (cnt >= 20.0), t, v20)
        v21 = jnp.where((c < 21.0) & (cnt >= 21.0), t, v21)
        return m, cnt, v20, v21

    zeros = jnp.zeros((_BB, 2 * _EMB), jnp.float32)
    inf = jnp.full((_BB, 2 * _EMB), jnp.inf, jnp.float32)
    _, _, v20, v21 = lax.fori_loop(
        0, 22, round_, (inf, zeros, zeros, zeros)
    )
    qs = v21 + jnp.float32(0.1) * (v20 - v21)
    s128 = jnp.sum(jnp.where(x0 >= qs[:, None, :], x0, 0.0), axis=1)
    pooled = dup(jnp.add, s128)[:, :_EMB] * jnp.float32(1.0 / _L)
    o_ref[...] = (
        jnp.dot(pooled, wt_ref[...], preferred_element_type=jnp.float32)
        + b_ref[...]
    )


def _tc_call(gathered3, wt, b2):
    grid = _B // _BB
    return pl.pallas_call(
        _tc_body,
        grid=(grid,),
        in_specs=[
            pl.BlockSpec((_BB, _LH, 2 * _EMB), lambda i: (i, 0, 0)),
            pl.BlockSpec((_EMB, _OUT), lambda i: (0, 0)),
            pl.BlockSpec((1, _OUT), lambda i: (0, 0)),
        ],
        out_specs=pl.BlockSpec((_BB, _OUT), lambda i: (i, 0)),
        out_shape=jax.ShapeDtypeStruct((_B, _OUT), jnp.float32),
    )(gathered3, wt, b2)


def kernel(tokens, table, W, b):
    tok2 = tokens.astype(jnp.int32).reshape(-1, _CH)
    gathered = _sc_gather(tok2, table)
    gathered3 = gathered.reshape(_B, _LH, 2 * _EMB)
    wt = W.T
    b2 = b.reshape(1, _OUT)
    return _tc_call(gathered3, wt, b2)


# speedup vs baseline: 5.8423x; 1.0028x over previous
import functools

import jax
import jax.numpy as jnp
from jax import lax
from jax.experimental import pallas as pl
from jax.experimental.pallas import tpu as pltpu
from jax.experimental.pallas import tpu_sc as plsc

_B = 4096
_L = 200
_EMB = 64
_OUT = 128

_NW = 32
_CH = 128
_NBUF = 5
_NPIPE = 4
_BP = _B // _NPIPE
_CHUNKS = (_BP * _L) // (_NW * _CH)
_NGROUP = _CHUNKS // _NBUF

_BB = 128
_LH = _L // 2
_SENT = -1e35


def _sc_gather(tok2, table):
    mesh = plsc.VectorSubcoreMesh(core_axis_name="c", subcore_axis_name="s")

    @functools.partial(
        pl.kernel,
        mesh=mesh,
        out_type=jax.ShapeDtypeStruct((_BP * _L, _EMB), jnp.float32),
        scratch_types=(
            [pltpu.VMEM((_CHUNKS, _CH), jnp.int32),
             pltpu.VMEM((_NBUF, _CH, _EMB), jnp.float32)]
            + [pltpu.SemaphoreType.DMA] * _NBUF
        ),
        compiler_params=pltpu.CompilerParams(use_tc_tiling_on_sc=False),
    )
    def k(tok_hbm, table_hbm, out_hbm, idx_v, rows_v, *sems):
        wid = lax.axis_index("s") * 2 + lax.axis_index("c")
        pltpu.sync_copy(tok_hbm.at[pl.ds(wid * _CHUNKS, _CHUNKS)], idx_v)
        for t in range(_NBUF):
            pltpu.async_copy(table_hbm.at[idx_v.at[t]], rows_v.at[t], sems[t])

        def body(g, carry):
            for t in range(_NBUF):
                j = g * _NBUF + t
                pltpu.make_async_copy(
                    table_hbm.at[idx_v.at[j]], rows_v.at[t], sems[t]
                ).wait()
                pltpu.sync_copy(
                    rows_v.at[t],
                    out_hbm.at[pl.ds((wid * _CHUNKS + j) * _CH, _CH)],
                )

                @pl.when(g < _NGROUP - 1)
                def _():
                    pltpu.async_copy(
                        table_hbm.at[idx_v.at[j + _NBUF]], rows_v.at[t], sems[t]
                    )

            return carry

        lax.fori_loop(0, _NGROUP, body, 0)

    return k(tok2, table)


def _tc_body(x_ref, wt_ref, b_ref, o_ref):
    x0 = x_ref[...]
    sent = jnp.float32(_SENT)

    def dup(op, a128):
        return op(a128, jnp.roll(a128, _EMB, axis=-1))

    def round_(_, carry):
        t, c, v20, v21 = carry
        masked = jnp.where(x0 < t[:, None, :], x0, sent)
        m = dup(jnp.maximum, jnp.max(masked, axis=1))
        s = dup(jnp.add, jnp.sum(masked, axis=1))
        cnt = jnp.round(s * jnp.float32(-1e-35))
        v20 = jnp.where((c < 20.0) & (cnt >= 20.0), t, v20)
        v21 = jnp.where((c < 21.0) & (cnt >= 21.0), t, v21)
        return m, cnt, v20, v21

    zeros = jnp.zeros((_BB, 2 * _EMB), jnp.float32)
    inf = jnp.full((_BB, 2 * _EMB), jnp.inf, jnp.float32)
    _, _, v20, v21 = lax.fori_loop(
        0, 22, round_, (inf, zeros, zeros, zeros)
    )
    qs = v21 + jnp.float32(0.1) * (v20 - v21)
    s128 = jnp.sum(jnp.where(x0 >= qs[:, None, :], x0, 0.0), axis=1)
    pooled = dup(jnp.add, s128)[:, :_EMB] * jnp.float32(1.0 / _L)
    o_ref[...] = (
        jnp.dot(pooled, wt_ref[...], preferred_element_type=jnp.float32)
        + b_ref[...]
    )


def _tc_call(gathered3, wt, b2):
    grid = _BP // _BB
    return pl.pallas_call(
        _tc_body,
        grid=(grid,),
        in_specs=[
            pl.BlockSpec((_BB, _LH, 2 * _EMB), lambda i: (i, 0, 0)),
            pl.BlockSpec((_EMB, _OUT), lambda i: (0, 0)),
            pl.BlockSpec((1, _OUT), lambda i: (0, 0)),
        ],
        out_specs=pl.BlockSpec((_BB, _OUT), lambda i: (i, 0)),
        out_shape=jax.ShapeDtypeStruct((_BP, _OUT), jnp.float32),
    )(gathered3, wt, b2)


def kernel(tokens, table, W, b):
    tok2 = tokens.astype(jnp.int32).reshape(-1, _CH)
    wt = W.T
    b2 = b.reshape(1, _OUT)
    rows_per_slice = (_BP * _L) // _CH
    outs = []
    for p in range(_NPIPE):
        tok_p = lax.slice_in_dim(tok2, p * rows_per_slice,
                                 (p + 1) * rows_per_slice, axis=0)
        gathered = _sc_gather(tok_p, table)
        gathered3 = gathered.reshape(_BP, _LH, 2 * _EMB)
        outs.append(_tc_call(gathered3, wt, b2))
    return jnp.concatenate(outs, axis=0)
